# baseline (speedup 1.0000x reference)
import functools

import jax
import jax.numpy as jnp
from jax import lax
from jax.experimental import pallas as pl
from jax.experimental.pallas import tpu as pltpu
from jax.experimental.pallas import tpu_sc as plsc

N = 10000
E = 320000
D = 128

NC = 2
NS = 16
NW = NC * NS
EP = E // NW
C = 80
NCH = EP // C
RP = N // NS
HD = D // 2
NP = 10240
DP = NP // NS

BN = 2000
GRID = N // BN

_f32 = jnp.float32
_i32 = jnp.int32

_SC_PARAMS = pltpu.CompilerParams(
    needs_layout_passes=False, use_tc_tiling_on_sc=False)


def _head_body(x_ref, w_ref, asr_ref, adr_ref,
               ha_ref, hb_ref, s_ref, d_ref, gm_ref):
    h = lax.dot(x_ref[...], w_ref[...], precision=lax.Precision.HIGHEST,
                preferred_element_type=_f32)
    ha_ref[...] = h[:, :HD]
    hb_ref[...] = h[:, HD:]
    s = lax.dot(h, asr_ref[...], precision=lax.Precision.HIGHEST,
                preferred_element_type=_f32)
    s_ref[...] = s
    d_ref[...] = lax.dot(h, adr_ref[...], precision=lax.Precision.HIGHEST,
                         preferred_element_type=_f32)

    @pl.when(pl.program_id(0) == 0)
    def _():
        gm_ref[...] = jnp.full((8, 128), -jnp.inf, _f32)
    gm_ref[...] = jnp.maximum(gm_ref[...], jnp.max(s))


def _tc_head(x, w, a_src, a_dst):
    return pl.pallas_call(
        _head_body,
        grid=(GRID,),
        in_specs=[
            pl.BlockSpec((BN, D), lambda i: (i, 0)),
            pl.BlockSpec((D, D), lambda i: (0, 0)),
            pl.BlockSpec((D, 1), lambda i: (0, 0)),
            pl.BlockSpec((D, 1), lambda i: (0, 0)),
        ],
        out_specs=[
            pl.BlockSpec((BN, HD), lambda i: (i, 0)),
            pl.BlockSpec((BN, HD), lambda i: (i, 0)),
            pl.BlockSpec((BN, 1), lambda i: (i, 0)),
            pl.BlockSpec((BN, 1), lambda i: (i, 0)),
            pl.BlockSpec((8, D), lambda i: (0, 0)),
        ],
        out_shape=[
            jax.ShapeDtypeStruct((N, HD), _f32),
            jax.ShapeDtypeStruct((N, HD), _f32),
            jax.ShapeDtypeStruct((N, 1), _f32),
            jax.ShapeDtypeStruct((N, 1), _f32),
            jax.ShapeDtypeStruct((8, D), _f32),
        ],
    )(x, w, a_src.reshape(D, 1), a_dst.reshape(D, 1))


def _assemble(o_ref, b_ref):
    x = jnp.concatenate(
        [o_ref[0, 0] + o_ref[1, 0], o_ref[0, 1] + o_ref[1, 1]], axis=1)
    x = x + b_ref[...]
    return jnp.where(x >= 0, x, 0.01 * x)


def _combhead_body(o_ref, b_ref, w_ref, asr_ref, adr_ref,
                   ha_ref, hb_ref, s_ref, d_ref, gm_ref):
    x1 = _assemble(o_ref, b_ref)
    h = lax.dot(x1, w_ref[...], precision=lax.Precision.HIGHEST,
                preferred_element_type=_f32)
    ha_ref[...] = h[:, :HD]
    hb_ref[...] = h[:, HD:]
    s = lax.dot(h, asr_ref[...], precision=lax.Precision.HIGHEST,
                preferred_element_type=_f32)
    s_ref[...] = s
    d_ref[...] = lax.dot(h, adr_ref[...], precision=lax.Precision.HIGHEST,
                         preferred_element_type=_f32)

    @pl.when(pl.program_id(0) == 0)
    def _():
        gm_ref[...] = jnp.full((8, 128), -jnp.inf, _f32)
    gm_ref[...] = jnp.maximum(gm_ref[...], jnp.max(s))


def _tc_combhead(o, b, w, a_src, a_dst):
    return pl.pallas_call(
        _combhead_body,
        grid=(GRID,),
        in_specs=[
            pl.BlockSpec((2, 2, BN, HD), lambda i: (0, 0, i, 0)),
            pl.BlockSpec((1, D), lambda i: (0, 0)),
            pl.BlockSpec((D, D), lambda i: (0, 0)),
            pl.BlockSpec((D, 1), lambda i: (0, 0)),
            pl.BlockSpec((D, 1), lambda i: (0, 0)),
        ],
        out_specs=[
            pl.BlockSpec((BN, HD), lambda i: (i, 0)),
            pl.BlockSpec((BN, HD), lambda i: (i, 0)),
            pl.BlockSpec((BN, 1), lambda i: (i, 0)),
            pl.BlockSpec((BN, 1), lambda i: (i, 0)),
            pl.BlockSpec((8, D), lambda i: (0, 0)),
        ],
        out_shape=[
            jax.ShapeDtypeStruct((N, HD), _f32),
            jax.ShapeDtypeStruct((N, HD), _f32),
            jax.ShapeDtypeStruct((N, 1), _f32),
            jax.ShapeDtypeStruct((N, 1), _f32),
            jax.ShapeDtypeStruct((8, D), _f32),
        ],
    )(o, b.reshape(1, D), w, a_src.reshape(D, 1), a_dst.reshape(D, 1))


def _comb_body(o_ref, b_ref, x_ref):
    x_ref[...] = _assemble(o_ref, b_ref)


def _tc_comb(o, b):
    return pl.pallas_call(
        _comb_body,
        grid=(GRID,),
        in_specs=[
            pl.BlockSpec((2, 2, BN, HD), lambda i: (0, 0, i, 0)),
            pl.BlockSpec((1, D), lambda i: (0, 0)),
        ],
        out_specs=pl.BlockSpec((BN, D), lambda i: (i, 0)),
        out_shape=jax.ShapeDtypeStruct((N, D), _f32),
    )(o, b.reshape(1, D))


_MESH = plsc.VectorSubcoreMesh(core_axis_name="c", subcore_axis_name="s")


@functools.partial(
    pl.kernel,
    out_type=(
        jax.ShapeDtypeStruct((E // C, C), _f32),
        jax.ShapeDtypeStruct((NC, NP), _f32),
    ),
    mesh=_MESH,
    compiler_params=_SC_PARAMS,
    scratch_types=[
        pltpu.VMEM((N,), _f32),
        pltpu.VMEM((N,), _f32),
        pltpu.VMEM((NCH, C), _i32),
        pltpu.VMEM((NCH, C), _i32),
        pltpu.VMEM((NCH, C), _f32),
        pltpu.VMEM((NP,), _f32),
        pltpu.VMEM((DP,), _f32),
        pltpu.VMEM((DP,), _f32),
        pltpu.VMEM((16,), _f32),
        pltpu.VMEM_SHARED((NS, NP), _f32),
    ],
)
def _sc_alpha(src2_hbm, dst2_hbm, as_hbm, ad_hbm, gm_hbm, p_hbm, den_hbm,
              as_v, ad_v, se_v, de_v, p_v, den_v, acc_v, tmp_v, gm_v,
              den_sh):
    c = lax.axis_index("c")
    s = lax.axis_index("s")
    wid = c * NS + s

    pltpu.sync_copy(as_hbm, as_v)
    pltpu.sync_copy(ad_hbm, ad_v)
    pltpu.sync_copy(gm_hbm, gm_v)
    pltpu.sync_copy(src2_hbm.at[pl.ds(wid * NCH, NCH)], se_v)
    pltpu.sync_copy(dst2_hbm.at[pl.ds(wid * NCH, NCH)], de_v)

    gmax = gm_v[...]

    zero16 = jnp.zeros((16,), _f32)

    def zbody(i, carry):
        den_v[pl.ds(i * 16, 16)] = zero16
        return carry
    lax.fori_loop(0, NP // 16, zbody, 0)

    def ebody(r, carry):
        for k in range(C // 16):
            sl = pl.ds(k * 16, 16)
            sv = se_v[r, sl]
            dv = de_v[r, sl]
            adg = plsc.load_gather(ad_v, [dv])
            av = plsc.load_gather(as_v, [sv]) + adg
            av = jnp.where(av >= 0, av, 0.2 * av)
            cv = gmax + adg
            cv = jnp.where(cv >= 0, cv, 0.2 * cv)
            p = jnp.exp(av - cv)
            p_v[r, sl] = p
            plsc.addupdate_scatter(den_v, [dv], p)
        return carry
    lax.fori_loop(0, NCH, ebody, 0)

    pltpu.sync_copy(p_v, p_hbm.at[pl.ds(wid * NCH, NCH)])

    pltpu.sync_copy(den_v, den_sh.at[s])
    plsc.subcore_barrier()

    col = pl.ds(s * DP, DP)
    pltpu.sync_copy(den_sh.at[0, col], acc_v)
    for j in range(1, NS):
        pltpu.sync_copy(den_sh.at[j, col], tmp_v)

        def rbody(i, carry, _j=j):
            sl = pl.ds(i * 16, 16)
            acc_v[sl] = acc_v[sl] + tmp_v[sl]
            return carry
        lax.fori_loop(0, DP // 16, rbody, 0)
    pltpu.sync_copy(acc_v, den_hbm.at[c, col])


@functools.partial(
    pl.kernel,
    out_type=(
        jax.ShapeDtypeStruct((E,), _f32),
        jax.ShapeDtypeStruct((NC, 2, N, HD), _f32),
    ),
    mesh=_MESH,
    compiler_params=_SC_PARAMS,
    scratch_types=[
        pltpu.VMEM((NCH, C), _i32),
        pltpu.VMEM((NCH, C), _i32),
        pltpu.VMEM((NCH, C), _f32),
        pltpu.VMEM((EP,), _f32),
        pltpu.VMEM((NP,), _f32),
        pltpu.VMEM((NP,), _f32),
        pltpu.VMEM((C, HD), _f32),
        pltpu.VMEM((C, HD), _f32),
        pltpu.VMEM((RP // 5, HD), _f32),
        pltpu.VMEM((C * 16,), _i32),
        pltpu.VMEM((C,), _f32),
        pltpu.VMEM_SHARED((N, HD), _f32),
        pltpu.SemaphoreType.DMA,
        pltpu.SemaphoreType.DMA,
    ],
)
def _sc_msg(src2_hbm, dst2_hbm, p_hbm, den_hbm, ha_hbm, hb_hbm, erep_hbm,
            w_hbm, out_hbm,
            se_v, de_v, p_v, w_v, den_v, denb_v, rows0_v, rows1_v, zb_v,
            er_v, wch_v, out_sh, sem0, sem1):
    c = lax.axis_index("c")
    s = lax.axis_index("s")
    wid = c * NS + s
    base = wid * EP

    pltpu.sync_copy(src2_hbm.at[pl.ds(wid * NCH, NCH)], se_v)
    pltpu.sync_copy(dst2_hbm.at[pl.ds(wid * NCH, NCH)], de_v)
    pltpu.sync_copy(p_hbm.at[pl.ds(wid * NCH, NCH)], p_v)
    pltpu.sync_copy(den_hbm.at[0], den_v)
    pltpu.sync_copy(den_hbm.at[1], denb_v)
    pltpu.sync_copy(erep_hbm, er_v)

    def dbody(i, carry):
        sl = pl.ds(i * 16, 16)
        den_v[sl] = den_v[sl] + denb_v[sl]
        return carry
    lax.fori_loop(0, NP // 16, dbody, 0)

    def wbody(r, carry):
        for k in range(C // 16):
            sl = pl.ds(k * 16, 16)
            dg = plsc.load_gather(den_v, [de_v[r, sl]])
            w_v[pl.ds(r * C + k * 16, 16)] = p_v[r, sl] / (dg + 1e-16)
        return carry
    lax.fori_loop(0, NCH, wbody, 0)

    pltpu.sync_copy(w_v, w_hbm.at[pl.ds(base, EP)])

    zero16 = jnp.zeros((16,), _f32)

    def zbody(i, carry):
        for g in range(HD // 16):
            zb_v[i, pl.ds(g * 16, 16)] = zero16
        return carry
    lax.fori_loop(0, RP // 5, zbody, 0)

    def _process(i, rv, sm, hh):
        pltpu.make_async_copy(hh.at[se_v.at[i]], rv, sm).wait()

        @plsc.parallel_loop(0, C // 16, 1, unroll=5)
        def _wcopy(k):
            wch_v[pl.ds(k * 16, 16)] = w_v[pl.ds(i * C + k * 16, 16)]

        @plsc.parallel_loop(0, C, 1, unroll=16)
        def _scale(e):
            wv = plsc.load_gather(wch_v, [er_v[pl.ds(e * 16, 16)]])
            for g in range(HD // 16):
                gsl = pl.ds(g * 16, 16)
                rv[e, gsl] = rv[e, gsl] * wv

        pltpu.sync_copy(rv, out_sh.at[de_v.at[i]], add=True)

    for half, hh_hbm in enumerate((ha_hbm, hb_hbm)):
        for k in range(5):
            pltpu.sync_copy(
                zb_v, out_sh.at[pl.ds(s * RP + k * (RP // 5), RP // 5)])
        plsc.subcore_barrier()

        pltpu.async_copy(hh_hbm.at[se_v.at[0]], rows0_v, sem0)
        pltpu.async_copy(hh_hbm.at[se_v.at[1]], rows1_v, sem1)

        def pair(g, carry, _hh=hh_hbm):
            for b, (rv, sm) in enumerate(((rows0_v, sem0),
                                          (rows1_v, sem1))):
                i = g * 2 + b
                _process(i, rv, sm, _hh)

                @pl.when(i + 2 < NCH)
                def _(_rv=rv, _sm=sm, _i=i):
                    pltpu.async_copy(_hh.at[se_v.at[_i + 2]], _rv, _sm)
            return carry
        lax.fori_loop(0, NCH // 2, pair, 0)
        _process(NCH - 1, rows0_v, sem0, hh_hbm)

        plsc.subcore_barrier()
        pltpu.sync_copy(out_sh.at[pl.ds(s * RP, RP)],
                        out_hbm.at[c, half, pl.ds(s * RP, RP)])


def _gat_layer(ha, hb, as_, ad, gm, src2d, dst2d, erep):
    p, den = _sc_alpha(src2d, dst2d, as_, ad, gm)
    w, o = _sc_msg(src2d, dst2d, p, den, ha, hb, erep)
    return w, o


def kernel(x, edge_index, W1, a_src1, a_dst1, b1, W2, a_src2, a_dst2, b2):
    src2d = edge_index[0].astype(_i32).reshape(E // C, C)
    dst2d = edge_index[1].astype(_i32).reshape(E // C, C)
    erep = jnp.repeat(jnp.arange(C, dtype=_i32), 16)

    ha1, hb1, s1, d1, gm1 = _tc_head(x, W1, a_src1, a_dst1)
    _, o1 = _gat_layer(ha1, hb1, s1.reshape(N), d1.reshape(N),
                       gm1[0, :16], src2d, dst2d, erep)
    ha2, hb2, s2, d2, gm2 = _tc_combhead(o1, b1, W2, a_src2, a_dst2)
    ew2, o2 = _gat_layer(ha2, hb2, s2.reshape(N), d2.reshape(N),
                         gm2[0, :16], src2d, dst2d, erep)
    x2 = _tc_comb(o2, b2)
    return (x2, (edge_index, ew2))


# speedup vs baseline: 1.1250x; 1.1250x over previous
import functools

import jax
import jax.numpy as jnp
from jax import lax
from jax.experimental import pallas as pl
from jax.experimental.pallas import tpu as pltpu
from jax.experimental.pallas import tpu_sc as plsc

N = 10000
E = 320000
D = 128

NC = 2
NS = 16
NW = NC * NS
EP = E // NW
C = 80
NCH = EP // C
RP = N // NS
HD = D // 2
NP = 10240
DP = NP // NS

BN = 2000
GRID = N // BN

_f32 = jnp.float32
_i32 = jnp.int32

_SC_PARAMS = pltpu.CompilerParams(
    needs_layout_passes=False, use_tc_tiling_on_sc=False)


def _head_body(x_ref, w_ref, asr_ref, adr_ref,
               ha_ref, hb_ref, s_ref, d_ref, gm_ref):
    h = lax.dot(x_ref[...], w_ref[...], precision=lax.Precision.HIGHEST,
                preferred_element_type=_f32)
    ha_ref[...] = h[:, :HD]
    hb_ref[...] = h[:, HD:]
    s = lax.dot(h, asr_ref[...], precision=lax.Precision.HIGHEST,
                preferred_element_type=_f32)
    s_ref[...] = s
    d_ref[...] = lax.dot(h, adr_ref[...], precision=lax.Precision.HIGHEST,
                         preferred_element_type=_f32)

    @pl.when(pl.program_id(0) == 0)
    def _():
        gm_ref[...] = jnp.full((8, 128), -jnp.inf, _f32)
    gm_ref[...] = jnp.maximum(gm_ref[...], jnp.max(s))


def _tc_head(x, w, a_src, a_dst):
    return pl.pallas_call(
        _head_body,
        grid=(GRID,),
        in_specs=[
            pl.BlockSpec((BN, D), lambda i: (i, 0)),
            pl.BlockSpec((D, D), lambda i: (0, 0)),
            pl.BlockSpec((D, 1), lambda i: (0, 0)),
            pl.BlockSpec((D, 1), lambda i: (0, 0)),
        ],
        out_specs=[
            pl.BlockSpec((BN, HD), lambda i: (i, 0)),
            pl.BlockSpec((BN, HD), lambda i: (i, 0)),
            pl.BlockSpec((BN, 1), lambda i: (i, 0)),
            pl.BlockSpec((BN, 1), lambda i: (i, 0)),
            pl.BlockSpec((8, D), lambda i: (0, 0)),
        ],
        out_shape=[
            jax.ShapeDtypeStruct((N, HD), _f32),
            jax.ShapeDtypeStruct((N, HD), _f32),
            jax.ShapeDtypeStruct((N, 1), _f32),
            jax.ShapeDtypeStruct((N, 1), _f32),
            jax.ShapeDtypeStruct((8, D), _f32),
        ],
    )(x, w, a_src.reshape(D, 1), a_dst.reshape(D, 1))


def _assemble(o_ref, b_ref):
    x = jnp.concatenate(
        [o_ref[0, 0] + o_ref[1, 0], o_ref[0, 1] + o_ref[1, 1]], axis=1)
    x = x + b_ref[...]
    return jnp.where(x >= 0, x, 0.01 * x)


def _combhead_body(o_ref, b_ref, w_ref, asr_ref, adr_ref,
                   ha_ref, hb_ref, s_ref, d_ref, gm_ref):
    x1 = _assemble(o_ref, b_ref)
    h = lax.dot(x1, w_ref[...], precision=lax.Precision.HIGHEST,
                preferred_element_type=_f32)
    ha_ref[...] = h[:, :HD]
    hb_ref[...] = h[:, HD:]
    s = lax.dot(h, asr_ref[...], precision=lax.Precision.HIGHEST,
                preferred_element_type=_f32)
    s_ref[...] = s
    d_ref[...] = lax.dot(h, adr_ref[...], precision=lax.Precision.HIGHEST,
                         preferred_element_type=_f32)

    @pl.when(pl.program_id(0) == 0)
    def _():
        gm_ref[...] = jnp.full((8, 128), -jnp.inf, _f32)
    gm_ref[...] = jnp.maximum(gm_ref[...], jnp.max(s))


def _tc_combhead(o, b, w, a_src, a_dst):
    return pl.pallas_call(
        _combhead_body,
        grid=(GRID,),
        in_specs=[
            pl.BlockSpec((2, 2, BN, HD), lambda i: (0, 0, i, 0)),
            pl.BlockSpec((1, D), lambda i: (0, 0)),
            pl.BlockSpec((D, D), lambda i: (0, 0)),
            pl.BlockSpec((D, 1), lambda i: (0, 0)),
            pl.BlockSpec((D, 1), lambda i: (0, 0)),
        ],
        out_specs=[
            pl.BlockSpec((BN, HD), lambda i: (i, 0)),
            pl.BlockSpec((BN, HD), lambda i: (i, 0)),
            pl.BlockSpec((BN, 1), lambda i: (i, 0)),
            pl.BlockSpec((BN, 1), lambda i: (i, 0)),
            pl.BlockSpec((8, D), lambda i: (0, 0)),
        ],
        out_shape=[
            jax.ShapeDtypeStruct((N, HD), _f32),
            jax.ShapeDtypeStruct((N, HD), _f32),
            jax.ShapeDtypeStruct((N, 1), _f32),
            jax.ShapeDtypeStruct((N, 1), _f32),
            jax.ShapeDtypeStruct((8, D), _f32),
        ],
    )(o, b.reshape(1, D), w, a_src.reshape(D, 1), a_dst.reshape(D, 1))


def _comb_body(o_ref, b_ref, x_ref):
    x_ref[...] = _assemble(o_ref, b_ref)


def _tc_comb(o, b):
    return pl.pallas_call(
        _comb_body,
        grid=(GRID,),
        in_specs=[
            pl.BlockSpec((2, 2, BN, HD), lambda i: (0, 0, i, 0)),
            pl.BlockSpec((1, D), lambda i: (0, 0)),
        ],
        out_specs=pl.BlockSpec((BN, D), lambda i: (i, 0)),
        out_shape=jax.ShapeDtypeStruct((N, D), _f32),
    )(o, b.reshape(1, D))


_MESH = plsc.VectorSubcoreMesh(core_axis_name="c", subcore_axis_name="s")


@functools.partial(
    pl.kernel,
    out_type=(
        jax.ShapeDtypeStruct((E // C, C), _f32),
        jax.ShapeDtypeStruct((NC, NP), _f32),
    ),
    mesh=_MESH,
    compiler_params=_SC_PARAMS,
    scratch_types=[
        pltpu.VMEM((N,), _f32),
        pltpu.VMEM((N,), _f32),
        pltpu.VMEM((NCH, C), _i32),
        pltpu.VMEM((NCH, C), _i32),
        pltpu.VMEM((NCH, C), _f32),
        pltpu.VMEM((NP,), _f32),
        pltpu.VMEM((DP,), _f32),
        pltpu.VMEM((DP,), _f32),
        pltpu.VMEM((16,), _f32),
        pltpu.VMEM_SHARED((NS, NP), _f32),
    ],
)
def _sc_alpha(src2_hbm, dst2_hbm, as_hbm, ad_hbm, gm_hbm, p_hbm, den_hbm,
              as_v, ad_v, se_v, de_v, p_v, den_v, acc_v, tmp_v, gm_v,
              den_sh):
    c = lax.axis_index("c")
    s = lax.axis_index("s")
    wid = c * NS + s

    pltpu.sync_copy(as_hbm, as_v)
    pltpu.sync_copy(ad_hbm, ad_v)
    pltpu.sync_copy(gm_hbm, gm_v)
    pltpu.sync_copy(src2_hbm.at[pl.ds(wid * NCH, NCH)], se_v)
    pltpu.sync_copy(dst2_hbm.at[pl.ds(wid * NCH, NCH)], de_v)

    gmax = gm_v[...]

    zero16 = jnp.zeros((16,), _f32)

    def zbody(i, carry):
        den_v[pl.ds(i * 16, 16)] = zero16
        return carry
    lax.fori_loop(0, NP // 16, zbody, 0)

    def ebody(r, carry):
        for k in range(C // 16):
            sl = pl.ds(k * 16, 16)
            sv = se_v[r, sl]
            dv = de_v[r, sl]
            adg = plsc.load_gather(ad_v, [dv])
            av = plsc.load_gather(as_v, [sv]) + adg
            av = jnp.where(av >= 0, av, 0.2 * av)
            cv = gmax + adg
            cv = jnp.where(cv >= 0, cv, 0.2 * cv)
            p = jnp.exp(av - cv)
            p_v[r, sl] = p
            plsc.addupdate_scatter(den_v, [dv], p)
        return carry
    lax.fori_loop(0, NCH, ebody, 0)

    pltpu.sync_copy(p_v, p_hbm.at[pl.ds(wid * NCH, NCH)])

    pltpu.sync_copy(den_v, den_sh.at[s])
    plsc.subcore_barrier()

    col = pl.ds(s * DP, DP)
    pltpu.sync_copy(den_sh.at[0, col], acc_v)
    for j in range(1, NS):
        pltpu.sync_copy(den_sh.at[j, col], tmp_v)

        def rbody(i, carry, _j=j):
            sl = pl.ds(i * 16, 16)
            acc_v[sl] = acc_v[sl] + tmp_v[sl]
            return carry
        lax.fori_loop(0, DP // 16, rbody, 0)
    pltpu.sync_copy(acc_v, den_hbm.at[c, col])


@functools.partial(
    pl.kernel,
    out_type=(
        jax.ShapeDtypeStruct((E,), _f32),
        jax.ShapeDtypeStruct((NC, 2, N, HD), _f32),
    ),
    mesh=_MESH,
    compiler_params=_SC_PARAMS,
    scratch_types=[
        pltpu.VMEM((NCH, C), _i32),
        pltpu.VMEM((NCH, C), _i32),
        pltpu.VMEM((NCH, C), _f32),
        pltpu.VMEM((EP,), _f32),
        pltpu.VMEM((NP,), _f32),
        pltpu.VMEM((NP,), _f32),
        pltpu.VMEM((C, HD), _f32),
        pltpu.VMEM((C, HD), _f32),
        pltpu.VMEM((C, HD), _f32),
        pltpu.VMEM((RP // 5, HD), _f32),
        pltpu.VMEM((C * 16,), _i32),
        pltpu.VMEM((C,), _f32),
        pltpu.VMEM_SHARED((N, HD), _f32),
        pltpu.SemaphoreType.DMA,
        pltpu.SemaphoreType.DMA,
        pltpu.SemaphoreType.DMA,
        pltpu.SemaphoreType.DMA,
        pltpu.SemaphoreType.DMA,
        pltpu.SemaphoreType.DMA,
    ],
)
def _sc_msg(src2_hbm, dst2_hbm, p_hbm, den_hbm, ha_hbm, hb_hbm, erep_hbm,
            w_hbm, out_hbm,
            se_v, de_v, p_v, w_v, den_v, denb_v, rows0_v, rows1_v, rows2_v,
            zb_v, er_v, wch_v, out_sh, sg0, sg1, sg2, ss0, ss1, ss2):
    c = lax.axis_index("c")
    s = lax.axis_index("s")
    wid = c * NS + s
    base = wid * EP

    pltpu.sync_copy(src2_hbm.at[pl.ds(wid * NCH, NCH)], se_v)
    pltpu.sync_copy(dst2_hbm.at[pl.ds(wid * NCH, NCH)], de_v)
    pltpu.sync_copy(p_hbm.at[pl.ds(wid * NCH, NCH)], p_v)
    pltpu.sync_copy(den_hbm.at[0], den_v)
    pltpu.sync_copy(den_hbm.at[1], denb_v)
    pltpu.sync_copy(erep_hbm, er_v)

    @plsc.parallel_loop(0, NP // 16, 1, unroll=8)
    def _dbody(i):
        sl = pl.ds(i * 16, 16)
        den_v[sl] = den_v[sl] + denb_v[sl]

    @plsc.parallel_loop(0, NCH, 1, unroll=4)
    def _wbody(r):
        for k in range(C // 16):
            sl = pl.ds(k * 16, 16)
            dg = plsc.load_gather(den_v, [de_v[r, sl]])
            w_v[pl.ds(r * C + k * 16, 16)] = p_v[r, sl] / (dg + 1e-16)

    pltpu.sync_copy(w_v, w_hbm.at[pl.ds(base, EP)])

    zero16 = jnp.zeros((16,), _f32)

    @plsc.parallel_loop(0, RP // 5, 1, unroll=8)
    def _zbody(i):
        for g in range(HD // 16):
            zb_v[i, pl.ds(g * 16, 16)] = zero16

    BUFS = ((rows0_v, sg0, ss0), (rows1_v, sg1, ss1), (rows2_v, sg2, ss2))

    def _stage(i, rv, sg, ss, hh):
        pltpu.make_async_copy(hh.at[se_v.at[i]], rv, sg).wait()

        @plsc.parallel_loop(0, C // 16, 1, unroll=5)
        def _wcopy(k):
            wch_v[pl.ds(k * 16, 16)] = w_v[pl.ds(i * C + k * 16, 16)]

        @plsc.parallel_loop(0, C, 1, unroll=16)
        def _scale(e):
            wv = plsc.load_gather(wch_v, [er_v[pl.ds(e * 16, 16)]])
            for g in range(HD // 16):
                gsl = pl.ds(g * 16, 16)
                rv[e, gsl] = rv[e, gsl] * wv

        pltpu.async_copy(rv, out_sh.at[de_v.at[i]], ss, add=True)

    for half, hh_hbm in enumerate((ha_hbm, hb_hbm)):
        for k in range(5):
            pltpu.sync_copy(
                zb_v, out_sh.at[pl.ds(s * RP + k * (RP // 5), RP // 5)])
        plsc.subcore_barrier()

        pltpu.async_copy(hh_hbm.at[se_v.at[0]], rows0_v, sg0)
        pltpu.async_copy(hh_hbm.at[se_v.at[1]], rows1_v, sg1)

        _stage(0, rows0_v, sg0, ss0, hh_hbm)
        pltpu.async_copy(hh_hbm.at[se_v.at[2]], rows2_v, sg2)

        def group(g, carry, _hh=hh_hbm):
            for b in range(3):
                j = 1 + g * 3 + b
                rv, sg, ss = BUFS[(1 + b) % 3]
                _stage(j, rv, sg, ss, _hh)
                nrv, nsg, nss = BUFS[b]

                @pl.when(j + 2 < NCH)
                def _(_hh=_hh, _j=j, _nrv=nrv, _nsg=nsg, _nss=nss):
                    pltpu.make_async_copy(
                        _nrv, out_sh.at[de_v.at[_j - 1]], _nss).wait()
                    pltpu.async_copy(_hh.at[se_v.at[_j + 2]], _nrv, _nsg)
            return carry
        lax.fori_loop(0, (NCH - 2) // 3, group, 0)
        _stage(NCH - 1, *BUFS[(NCH - 1) % 3], hh_hbm)

        for j in (NCH - 3, NCH - 2, NCH - 1):
            rv, sg, ss = BUFS[j % 3]
            pltpu.make_async_copy(rv, out_sh.at[de_v.at[j]], ss).wait()

        plsc.subcore_barrier()
        pltpu.sync_copy(out_sh.at[pl.ds(s * RP, RP)],
                        out_hbm.at[c, half, pl.ds(s * RP, RP)])


def _gat_layer(ha, hb, as_, ad, gm, src2d, dst2d, erep):
    p, den = _sc_alpha(src2d, dst2d, as_, ad, gm)
    w, o = _sc_msg(src2d, dst2d, p, den, ha, hb, erep)
    return w, o


def kernel(x, edge_index, W1, a_src1, a_dst1, b1, W2, a_src2, a_dst2, b2):
    src2d = edge_index[0].astype(_i32).reshape(E // C, C)
    dst2d = edge_index[1].astype(_i32).reshape(E // C, C)
    erep = jnp.repeat(jnp.arange(C, dtype=_i32), 16)

    ha1, hb1, s1, d1, gm1 = _tc_head(x, W1, a_src1, a_dst1)
    _, o1 = _gat_layer(ha1, hb1, s1.reshape(N), d1.reshape(N),
                       gm1[0, :16], src2d, dst2d, erep)
    ha2, hb2, s2, d2, gm2 = _tc_combhead(o1, b1, W2, a_src2, a_dst2)
    ew2, o2 = _gat_layer(ha2, hb2, s2.reshape(N), d2.reshape(N),
                         gm2[0, :16], src2d, dst2d, erep)
    x2 = _tc_comb(o2, b2)
    return (x2, (edge_index, ew2))


# speedup vs baseline: 1.1917x; 1.0592x over previous
import functools

import jax
import jax.numpy as jnp
from jax import lax
from jax.experimental import pallas as pl
from jax.experimental.pallas import tpu as pltpu
from jax.experimental.pallas import tpu_sc as plsc

N = 10000
E = 320000
D = 128

NC = 2
NS = 16
NW = NC * NS
EP = E // NW
C = 80
NCH = EP // C
RP = N // NS
HD = D // 2
NP = 10240
DP = NP // NS

BN = 2000
GRID = N // BN

_f32 = jnp.float32
_i32 = jnp.int32

_SC_PARAMS = pltpu.CompilerParams(
    needs_layout_passes=False, use_tc_tiling_on_sc=False)


def _head_body(x_ref, w_ref, asr_ref, adr_ref,
               ha_ref, hb_ref, s_ref, d_ref, gm_ref):
    h = lax.dot(x_ref[...], w_ref[...], precision=lax.Precision.HIGHEST,
                preferred_element_type=_f32)
    ha_ref[...] = h[:, :HD]
    hb_ref[...] = h[:, HD:]
    s = lax.dot(h, asr_ref[...], precision=lax.Precision.HIGHEST,
                preferred_element_type=_f32)
    s_ref[...] = s
    d_ref[...] = lax.dot(h, adr_ref[...], precision=lax.Precision.HIGHEST,
                         preferred_element_type=_f32)

    @pl.when(pl.program_id(0) == 0)
    def _():
        gm_ref[...] = jnp.full((8, 128), -jnp.inf, _f32)
    gm_ref[...] = jnp.maximum(gm_ref[...], jnp.max(s))


def _tc_head(x, w, a_src, a_dst):
    return pl.pallas_call(
        _head_body,
        grid=(GRID,),
        in_specs=[
            pl.BlockSpec((BN, D), lambda i: (i, 0)),
            pl.BlockSpec((D, D), lambda i: (0, 0)),
            pl.BlockSpec((D, 1), lambda i: (0, 0)),
            pl.BlockSpec((D, 1), lambda i: (0, 0)),
        ],
        out_specs=[
            pl.BlockSpec((BN, HD), lambda i: (i, 0)),
            pl.BlockSpec((BN, HD), lambda i: (i, 0)),
            pl.BlockSpec((BN, 1), lambda i: (i, 0)),
            pl.BlockSpec((BN, 1), lambda i: (i, 0)),
            pl.BlockSpec((8, D), lambda i: (0, 0)),
        ],
        out_shape=[
            jax.ShapeDtypeStruct((N, HD), _f32),
            jax.ShapeDtypeStruct((N, HD), _f32),
            jax.ShapeDtypeStruct((N, 1), _f32),
            jax.ShapeDtypeStruct((N, 1), _f32),
            jax.ShapeDtypeStruct((8, D), _f32),
        ],
    )(x, w, a_src.reshape(D, 1), a_dst.reshape(D, 1))


def _assemble(o_ref, b_ref):
    x = jnp.concatenate(
        [o_ref[0, 0] + o_ref[1, 0], o_ref[0, 1] + o_ref[1, 1]], axis=1)
    x = x + b_ref[...]
    return jnp.where(x >= 0, x, 0.01 * x)


def _combhead_body(o_ref, b_ref, w_ref, asr_ref, adr_ref,
                   ha_ref, hb_ref, s_ref, d_ref, gm_ref):
    x1 = _assemble(o_ref, b_ref)
    h = lax.dot(x1, w_ref[...], precision=lax.Precision.HIGHEST,
                preferred_element_type=_f32)
    ha_ref[...] = h[:, :HD]
    hb_ref[...] = h[:, HD:]
    s = lax.dot(h, asr_ref[...], precision=lax.Precision.HIGHEST,
                preferred_element_type=_f32)
    s_ref[...] = s
    d_ref[...] = lax.dot(h, adr_ref[...], precision=lax.Precision.HIGHEST,
                         preferred_element_type=_f32)

    @pl.when(pl.program_id(0) == 0)
    def _():
        gm_ref[...] = jnp.full((8, 128), -jnp.inf, _f32)
    gm_ref[...] = jnp.maximum(gm_ref[...], jnp.max(s))


def _tc_combhead(o, b, w, a_src, a_dst):
    return pl.pallas_call(
        _combhead_body,
        grid=(GRID,),
        in_specs=[
            pl.BlockSpec((2, 2, BN, HD), lambda i: (0, 0, i, 0)),
            pl.BlockSpec((1, D), lambda i: (0, 0)),
            pl.BlockSpec((D, D), lambda i: (0, 0)),
            pl.BlockSpec((D, 1), lambda i: (0, 0)),
            pl.BlockSpec((D, 1), lambda i: (0, 0)),
        ],
        out_specs=[
            pl.BlockSpec((BN, HD), lambda i: (i, 0)),
            pl.BlockSpec((BN, HD), lambda i: (i, 0)),
            pl.BlockSpec((BN, 1), lambda i: (i, 0)),
            pl.BlockSpec((BN, 1), lambda i: (i, 0)),
            pl.BlockSpec((8, D), lambda i: (0, 0)),
        ],
        out_shape=[
            jax.ShapeDtypeStruct((N, HD), _f32),
            jax.ShapeDtypeStruct((N, HD), _f32),
            jax.ShapeDtypeStruct((N, 1), _f32),
            jax.ShapeDtypeStruct((N, 1), _f32),
            jax.ShapeDtypeStruct((8, D), _f32),
        ],
    )(o, b.reshape(1, D), w, a_src.reshape(D, 1), a_dst.reshape(D, 1))


def _comb_body(o_ref, b_ref, x_ref):
    x_ref[...] = _assemble(o_ref, b_ref)


def _tc_comb(o, b):
    return pl.pallas_call(
        _comb_body,
        grid=(GRID,),
        in_specs=[
            pl.BlockSpec((2, 2, BN, HD), lambda i: (0, 0, i, 0)),
            pl.BlockSpec((1, D), lambda i: (0, 0)),
        ],
        out_specs=pl.BlockSpec((BN, D), lambda i: (i, 0)),
        out_shape=jax.ShapeDtypeStruct((N, D), _f32),
    )(o, b.reshape(1, D))


_MESH = plsc.VectorSubcoreMesh(core_axis_name="c", subcore_axis_name="s")


@functools.partial(
    pl.kernel,
    out_type=(
        jax.ShapeDtypeStruct((E // C, C), _f32),
        jax.ShapeDtypeStruct((NC, NP), _f32),
    ),
    mesh=_MESH,
    compiler_params=_SC_PARAMS,
    scratch_types=[
        pltpu.VMEM((N,), _f32),
        pltpu.VMEM((N,), _f32),
        pltpu.VMEM((NCH, C), _i32),
        pltpu.VMEM((NCH, C), _i32),
        pltpu.VMEM((NCH, C), _f32),
        pltpu.VMEM((NP,), _f32),
        pltpu.VMEM((DP,), _f32),
        pltpu.VMEM((DP,), _f32),
        pltpu.VMEM((16,), _f32),
        pltpu.VMEM_SHARED((NS, NP), _f32),
    ],
)
def _sc_alpha(src2_hbm, dst2_hbm, as_hbm, ad_hbm, gm_hbm, p_hbm, den_hbm,
              as_v, ad_v, se_v, de_v, p_v, den_v, acc_v, tmp_v, gm_v,
              den_sh):
    c = lax.axis_index("c")
    s = lax.axis_index("s")
    wid = c * NS + s

    pltpu.sync_copy(as_hbm, as_v)
    pltpu.sync_copy(ad_hbm, ad_v)
    pltpu.sync_copy(gm_hbm, gm_v)
    pltpu.sync_copy(src2_hbm.at[pl.ds(wid * NCH, NCH)], se_v)
    pltpu.sync_copy(dst2_hbm.at[pl.ds(wid * NCH, NCH)], de_v)

    gmax = gm_v[...]

    zero16 = jnp.zeros((16,), _f32)

    @plsc.parallel_loop(0, NP // 16, 1, unroll=8)
    def _zdbody(i):
        den_v[pl.ds(i * 16, 16)] = zero16

    @plsc.parallel_loop(0, NCH, 1, unroll=4)
    def _ebody(r):
        for k in range(C // 16):
            sl = pl.ds(k * 16, 16)
            sv = se_v[r, sl]
            dv = de_v[r, sl]
            adg = plsc.load_gather(ad_v, [dv])
            av = plsc.load_gather(as_v, [sv]) + adg
            av = jnp.where(av >= 0, av, 0.2 * av)
            cv = gmax + adg
            cv = jnp.where(cv >= 0, cv, 0.2 * cv)
            p = jnp.exp(av - cv)
            p_v[r, sl] = p
            plsc.addupdate_scatter(den_v, [dv], p)

    pltpu.sync_copy(p_v, p_hbm.at[pl.ds(wid * NCH, NCH)])

    pltpu.sync_copy(den_v, den_sh.at[s])
    plsc.subcore_barrier()

    col = pl.ds(s * DP, DP)
    pltpu.sync_copy(den_sh.at[0, col], acc_v)
    for j in range(1, NS):
        pltpu.sync_copy(den_sh.at[j, col], tmp_v)

        @plsc.parallel_loop(0, DP // 16, 1, unroll=8)
        def _rbody(i, _j=j):
            sl = pl.ds(i * 16, 16)
            acc_v[sl] = acc_v[sl] + tmp_v[sl]
    pltpu.sync_copy(acc_v, den_hbm.at[c, col])


@functools.partial(
    pl.kernel,
    out_type=(
        jax.ShapeDtypeStruct((E,), _f32),
        jax.ShapeDtypeStruct((NC, 2, N, HD), _f32),
    ),
    mesh=_MESH,
    compiler_params=_SC_PARAMS,
    scratch_types=[
        pltpu.VMEM((NCH, C), _i32),
        pltpu.VMEM((NCH, C), _i32),
        pltpu.VMEM((NCH, C), _f32),
        pltpu.VMEM((EP,), _f32),
        pltpu.VMEM((NP,), _f32),
        pltpu.VMEM((NP,), _f32),
        pltpu.VMEM((C, HD), _f32),
        pltpu.VMEM((C, HD), _f32),
        pltpu.VMEM((C, HD), _f32),
        pltpu.VMEM((RP // 5, HD), _f32),
        pltpu.VMEM((C * 16,), _i32),
        pltpu.VMEM((C,), _f32),
        pltpu.VMEM_SHARED((N, HD), _f32),
        pltpu.SemaphoreType.DMA,
        pltpu.SemaphoreType.DMA,
        pltpu.SemaphoreType.DMA,
        pltpu.SemaphoreType.DMA,
        pltpu.SemaphoreType.DMA,
        pltpu.SemaphoreType.DMA,
    ],
)
def _sc_msg(src2_hbm, dst2_hbm, p_hbm, den_hbm, ha_hbm, hb_hbm, erep_hbm,
            w_hbm, out_hbm,
            se_v, de_v, p_v, w_v, den_v, denb_v, rows0_v, rows1_v, rows2_v,
            zb_v, er_v, wch_v, out_sh, sg0, sg1, sg2, ss0, ss1, ss2):
    c = lax.axis_index("c")
    s = lax.axis_index("s")
    wid = c * NS + s
    base = wid * EP

    pltpu.sync_copy(src2_hbm.at[pl.ds(wid * NCH, NCH)], se_v)
    pltpu.sync_copy(dst2_hbm.at[pl.ds(wid * NCH, NCH)], de_v)
    pltpu.sync_copy(p_hbm.at[pl.ds(wid * NCH, NCH)], p_v)
    pltpu.sync_copy(den_hbm.at[0], den_v)
    pltpu.sync_copy(den_hbm.at[1], denb_v)
    pltpu.sync_copy(erep_hbm, er_v)

    @plsc.parallel_loop(0, NP // 16, 1, unroll=8)
    def _dbody(i):
        sl = pl.ds(i * 16, 16)
        den_v[sl] = den_v[sl] + denb_v[sl]

    @plsc.parallel_loop(0, NCH, 1, unroll=4)
    def _wbody(r):
        for k in range(C // 16):
            sl = pl.ds(k * 16, 16)
            dg = plsc.load_gather(den_v, [de_v[r, sl]])
            w_v[pl.ds(r * C + k * 16, 16)] = p_v[r, sl] / (dg + 1e-16)

    pltpu.sync_copy(w_v, w_hbm.at[pl.ds(base, EP)])

    zero16 = jnp.zeros((16,), _f32)

    @plsc.parallel_loop(0, RP // 5, 1, unroll=8)
    def _zbody(i):
        for g in range(HD // 16):
            zb_v[i, pl.ds(g * 16, 16)] = zero16

    BUFS = ((rows0_v, sg0, ss0), (rows1_v, sg1, ss1), (rows2_v, sg2, ss2))

    def _stage(i, rv, sg, ss, hh):
        pltpu.make_async_copy(hh.at[se_v.at[i]], rv, sg).wait()

        @plsc.parallel_loop(0, C // 16, 1, unroll=5)
        def _wcopy(k):
            wch_v[pl.ds(k * 16, 16)] = w_v[pl.ds(i * C + k * 16, 16)]

        @plsc.parallel_loop(0, C, 1, unroll=16)
        def _scale(e):
            wv = plsc.load_gather(wch_v, [er_v[pl.ds(e * 16, 16)]])
            for g in range(HD // 16):
                gsl = pl.ds(g * 16, 16)
                rv[e, gsl] = rv[e, gsl] * wv

        pltpu.async_copy(rv, out_sh.at[de_v.at[i]], ss, add=True)

    for half, hh_hbm in enumerate((ha_hbm, hb_hbm)):
        for k in range(5):
            pltpu.sync_copy(
                zb_v, out_sh.at[pl.ds(s * RP + k * (RP // 5), RP // 5)])
        plsc.subcore_barrier()

        pltpu.async_copy(hh_hbm.at[se_v.at[0]], rows0_v, sg0)
        pltpu.async_copy(hh_hbm.at[se_v.at[1]], rows1_v, sg1)

        _stage(0, rows0_v, sg0, ss0, hh_hbm)
        pltpu.async_copy(hh_hbm.at[se_v.at[2]], rows2_v, sg2)

        def group(g, carry, _hh=hh_hbm):
            for b in range(3):
                j = 1 + g * 3 + b
                rv, sg, ss = BUFS[(1 + b) % 3]
                _stage(j, rv, sg, ss, _hh)
                nrv, nsg, nss = BUFS[b]

                @pl.when(j + 2 < NCH)
                def _(_hh=_hh, _j=j, _nrv=nrv, _nsg=nsg, _nss=nss):
                    pltpu.make_async_copy(
                        _nrv, out_sh.at[de_v.at[_j - 1]], _nss).wait()
                    pltpu.async_copy(_hh.at[se_v.at[_j + 2]], _nrv, _nsg)
            return carry
        lax.fori_loop(0, (NCH - 2) // 3, group, 0)
        _stage(NCH - 1, *BUFS[(NCH - 1) % 3], hh_hbm)

        for j in (NCH - 3, NCH - 2, NCH - 1):
            rv, sg, ss = BUFS[j % 3]
            pltpu.make_async_copy(rv, out_sh.at[de_v.at[j]], ss).wait()

        plsc.subcore_barrier()
        pltpu.sync_copy(out_sh.at[pl.ds(s * RP, RP)],
                        out_hbm.at[c, half, pl.ds(s * RP, RP)])


def _gat_layer(ha, hb, as_, ad, gm, src2d, dst2d, erep):
    p, den = _sc_alpha(src2d, dst2d, as_, ad, gm)
    w, o = _sc_msg(src2d, dst2d, p, den, ha, hb, erep)
    return w, o


def kernel(x, edge_index, W1, a_src1, a_dst1, b1, W2, a_src2, a_dst2, b2):
    src2d = edge_index[0].astype(_i32).reshape(E // C, C)
    dst2d = edge_index[1].astype(_i32).reshape(E // C, C)
    erep = jnp.repeat(jnp.arange(C, dtype=_i32), 16)

    ha1, hb1, s1, d1, gm1 = _tc_head(x, W1, a_src1, a_dst1)
    _, o1 = _gat_layer(ha1, hb1, s1.reshape(N), d1.reshape(N),
                       gm1[0, :16], src2d, dst2d, erep)
    ha2, hb2, s2, d2, gm2 = _tc_combhead(o1, b1, W2, a_src2, a_dst2)
    ew2, o2 = _gat_layer(ha2, hb2, s2.reshape(N), d2.reshape(N),
                         gm2[0, :16], src2d, dst2d, erep)
    x2 = _tc_comb(o2, b2)
    return (x2, (edge_index, ew2))
